# baseline (speedup 1.0000x reference)
import functools

import jax
import jax.numpy as jnp
from jax import lax
from jax.experimental import pallas as pl
from jax.experimental.pallas import tpu as pltpu
from jax.experimental.pallas import tpu_sc as plsc

N_NODES = 10000
N_EDGES = 320000
D = 128

NC = 2
NS = 16
NW = NC * NS

CHUNK = 120
RK = 3
NR = 56
E_PAD = NW * NR * RK * CHUNK
N_PAD = 10240
ROWS_PER_TILE = N_PAD // NS
G = 2


def _sc_messages(x_pad, sdx, zeros_tile):
    mesh = plsc.VectorSubcoreMesh(core_axis_name="c", subcore_axis_name="s")

    @functools.partial(
        pl.kernel,
        out_type=jax.ShapeDtypeStruct((NC, N_PAD, D), jnp.float32),
        mesh=mesh,
        scratch_types=[
            pltpu.VMEM((3, 2, RK, CHUNK), jnp.int32),
            pltpu.VMEM((RK, CHUNK, D), jnp.float32),
            pltpu.VMEM_SHARED((N_PAD, D), jnp.float32),
            pltpu.SemaphoreType.DMA((3,)),
            pltpu.SemaphoreType.DMA((RK,)),
            pltpu.SemaphoreType.DMA((RK,)),
        ],
    )
    def body(x_hbm, sdx_hbm, zero_hbm, out_hbm, sdx_v, rows_v, acc,
             sem_i, sem_g, sem_s):
        cid = lax.axis_index("c")
        sid = lax.axis_index("s")
        wid = sid * NC + cid

        def fire_gather(p, j, slot):
            pltpu.async_copy(x_hbm.at[sdx_v.at[p, 0, j]], rows_v.at[slot],
                             sem_g.at[slot])

        def fire_scatter(p, k):
            pltpu.async_copy(rows_v.at[k], acc.at[sdx_v.at[p, 1, k]],
                             sem_s.at[k], add=True)

        def wait_rows_sem(sem, slot):
            pltpu.make_async_copy(x_hbm.at[pl.ds(0, CHUNK)], rows_v.at[slot],
                                  sem.at[slot]).wait()

        def wait_idx(p):
            pltpu.make_async_copy(sdx_hbm.at[0, 0], sdx_v.at[p],
                                  sem_i.at[p]).wait()

        def round_body(r, pi, first=False, last=False):
            pn = (pi + 1) % 3
            if not last:
                pltpu.async_copy(sdx_hbm.at[wid, r + 1], sdx_v.at[pn],
                                 sem_i.at[pn])
            for k in range(RK):
                gj = (k + G) % RK
                gp = pi if k < RK - G else pn
                if k == RK - G and not last:
                    wait_idx(pn)
                if not (last and k >= RK - G):
                    if not (first and k < RK - G):
                        wait_rows_sem(sem_s, gj)
                    fire_gather(gp, gj, gj)
                wait_rows_sem(sem_g, k)
                fire_scatter(pi, k)
            if last:
                for b in range(RK):
                    wait_rows_sem(sem_s, b)

        pltpu.sync_copy(zero_hbm,
                        acc.at[pl.ds(sid * ROWS_PER_TILE, ROWS_PER_TILE)])
        plsc.subcore_barrier()

        pltpu.sync_copy(sdx_hbm.at[wid, 0], sdx_v.at[0])
        for j in range(G):
            fire_gather(0, j, j)

        round_body(0, 0, first=True)

        @pl.loop(0, (NR - 2) // 3)
        def _(t):
            r = 1 + 3 * t
            round_body(r, 1)
            round_body(r + 1, 2)
            round_body(r + 2, 0)

        round_body(NR - 1, (NR - 1) % 3, last=True)

        plsc.subcore_barrier()
        pltpu.sync_copy(
            acc.at[pl.ds(sid * ROWS_PER_TILE, ROWS_PER_TILE)],
            out_hbm.at[cid, pl.ds(sid * ROWS_PER_TILE, ROWS_PER_TILE)],
        )

    return body(x_pad, sdx, zeros_tile)


def _tc_xw1b(x, W1b, b1):
    BN = 2048

    def body(x_ref, w1b_ref, b1_ref, o_ref):
        o_ref[...] = (
            jnp.dot(x_ref[...], w1b_ref[...], preferred_element_type=jnp.float32,
                    precision=lax.Precision.DEFAULT)
            + b1_ref[...]
        )

    full = lambda shape: pl.BlockSpec(shape, lambda i: tuple(0 for _ in shape))
    return pl.pallas_call(
        body,
        grid=(N_PAD // BN,),
        in_specs=[
            pl.BlockSpec((BN, D), lambda i: (i, 0)),
            full((D, D)),
            full((1, D)),
        ],
        out_specs=pl.BlockSpec((BN, D), lambda i: (i, 0)),
        out_shape=jax.ShapeDtypeStruct((N_PAD, D), jnp.float32),
    )(x, W1b, b1)


def _tc_mlp(msgs, xb, W1a, g1, be1, W2, b2):
    BN = 2048

    def body(m_ref, xb_ref, w1a_ref, g1_ref, be1_ref,
             w2_ref, b2_ref, o_ref):
        m = m_ref[0] + m_ref[1]
        h = (
            jnp.dot(m, w1a_ref[...], preferred_element_type=jnp.float32,
                    precision=lax.Precision.DEFAULT)
            + xb_ref[...]
        )
        mu = jnp.mean(h, axis=-1, keepdims=True)
        var = jnp.mean((h - mu) ** 2, axis=-1, keepdims=True)
        h = (h - mu) * lax.rsqrt(var + 1e-5) * g1_ref[...] + be1_ref[...]
        h = jnp.tanh(h)
        o_ref[...] = (
            jnp.dot(h, w2_ref[...], preferred_element_type=jnp.float32,
                    precision=lax.Precision.DEFAULT)
            + b2_ref[...]
        )

    full = lambda shape: pl.BlockSpec(shape, lambda i: tuple(0 for _ in shape))
    return pl.pallas_call(
        body,
        grid=(N_PAD // BN,),
        in_specs=[
            pl.BlockSpec((NC, BN, D), lambda i: (0, i, 0)),
            pl.BlockSpec((BN, D), lambda i: (i, 0)),
            full((D, D)),
            full((1, D)),
            full((1, D)),
            full((D, D)),
            full((1, D)),
        ],
        out_specs=pl.BlockSpec((BN, D), lambda i: (i, 0)),
        out_shape=jax.ShapeDtypeStruct((N_NODES, D), jnp.float32),
    )(msgs, xb, W1a, g1, be1, W2, b2)


def kernel(x, edge_index, W1, b1, g1, be1, W2, b2):
    x_pad = jnp.pad(x, ((0, N_PAD - N_NODES), (0, 0)))
    s = edge_index[0]
    e = edge_index[1]
    n_fill = E_PAD - 2 * N_EDGES
    fill = jnp.arange(n_fill, dtype=jnp.int32)
    src_fill = fill * 97 % N_NODES
    dst_fill = N_NODES + fill % (N_PAD - N_NODES)
    src = jnp.concatenate([s, e, src_fill]).reshape(NW, NR, RK, CHUNK)
    dst = jnp.concatenate([e, s, dst_fill]).reshape(NW, NR, RK, CHUNK)
    sdx = jnp.stack([src, dst], axis=2)
    zeros_tile = jnp.zeros((ROWS_PER_TILE, D), jnp.float32)

    msgs = _sc_messages(x_pad, sdx, zeros_tile)
    xb = _tc_xw1b(x_pad, W1[D:], b1.reshape(1, D))
    out = _tc_mlp(msgs, xb, W1[:D],
                  g1.reshape(1, D), be1.reshape(1, D), W2, b2.reshape(1, D))
    return out


# speedup vs baseline: 1.0111x; 1.0111x over previous
import functools

import jax
import jax.numpy as jnp
from jax import lax
from jax.experimental import pallas as pl
from jax.experimental.pallas import tpu as pltpu
from jax.experimental.pallas import tpu_sc as plsc

N_NODES = 10000
N_EDGES = 320000
D = 128

NC = 2
NS = 16
NW = NC * NS

CHUNK = 120
RK = 3
NR = 56
E_PAD = NW * NR * RK * CHUNK
N_PAD = 10240
ROWS_PER_TILE = N_PAD // NS
G = 2


def _sc_messages(x_pad, sdx, zeros_tile):
    mesh = plsc.VectorSubcoreMesh(core_axis_name="c", subcore_axis_name="s")

    @functools.partial(
        pl.kernel,
        out_type=jax.ShapeDtypeStruct((NC, N_PAD, D), jnp.float32),
        mesh=mesh,
        scratch_types=[
            pltpu.VMEM((3, 2, RK, CHUNK), jnp.int32),
            pltpu.VMEM((RK, CHUNK, D), jnp.float32),
            pltpu.VMEM_SHARED((N_PAD, D), jnp.float32),
            pltpu.SemaphoreType.DMA((3,)),
            pltpu.SemaphoreType.DMA((RK,)),
            pltpu.SemaphoreType.DMA((RK,)),
        ],
    )
    def body(x_hbm, sdx_hbm, zero_hbm, out_hbm, sdx_v, rows_v, acc,
             sem_i, sem_g, sem_s):
        cid = lax.axis_index("c")
        sid = lax.axis_index("s")
        wid = sid * NC + cid

        def fire_gather(p, j, slot):
            pltpu.async_copy(x_hbm.at[sdx_v.at[p, 0, j]], rows_v.at[slot],
                             sem_g.at[slot])

        def fire_scatter(p, k):
            pltpu.async_copy(rows_v.at[k], acc.at[sdx_v.at[p, 1, k]],
                             sem_s.at[k], add=True)

        def wait_rows_sem(sem, slot):
            pltpu.make_async_copy(x_hbm.at[pl.ds(0, CHUNK)], rows_v.at[slot],
                                  sem.at[slot]).wait()

        def wait_idx(p):
            pltpu.make_async_copy(sdx_hbm.at[0, 0], sdx_v.at[p],
                                  sem_i.at[p]).wait()

        def round_body(r, pi, first=False, last=False):
            pn = (pi + 1) % 3
            if not last:
                pltpu.async_copy(sdx_hbm.at[wid, r + 1], sdx_v.at[pn],
                                 sem_i.at[pn])
            for k in range(RK):
                gj = (k + G) % RK
                gp = pi if k < RK - G else pn
                if k == RK - G and not last:
                    wait_idx(pn)
                if not (last and k >= RK - G):
                    if not (first and k < RK - G):
                        wait_rows_sem(sem_s, gj)
                    fire_gather(gp, gj, gj)
                wait_rows_sem(sem_g, k)
                fire_scatter(pi, k)
            if last:
                for b in range(RK):
                    wait_rows_sem(sem_s, b)

        pltpu.sync_copy(zero_hbm,
                        acc.at[pl.ds(sid * ROWS_PER_TILE, ROWS_PER_TILE)])
        plsc.subcore_barrier()

        pltpu.sync_copy(sdx_hbm.at[wid, 0], sdx_v.at[0])
        for j in range(G):
            fire_gather(0, j, j)

        round_body(0, 0, first=True)

        @pl.loop(0, (NR - 2) // 3)
        def _(t):
            r = 1 + 3 * t
            round_body(r, 1)
            round_body(r + 1, 2)
            round_body(r + 2, 0)

        round_body(NR - 1, (NR - 1) % 3, last=True)

        plsc.subcore_barrier()
        pltpu.sync_copy(
            acc.at[pl.ds(sid * ROWS_PER_TILE, ROWS_PER_TILE)],
            out_hbm.at[cid, pl.ds(sid * ROWS_PER_TILE, ROWS_PER_TILE)],
        )

    return body(x_pad, sdx, zeros_tile)


def _tc_mlp(msgs, x_pad, W1a, W1b, b1, g1, be1, W2, b2):
    BN = 2048

    def body(m_ref, x_ref, w1a_ref, w1b_ref, b1_ref, g1_ref, be1_ref,
             w2_ref, b2_ref, o_ref):
        m = m_ref[0] + m_ref[1]
        h = (
            jnp.dot(m, w1a_ref[...], preferred_element_type=jnp.float32,
                    precision=lax.Precision.DEFAULT)
            + jnp.dot(x_ref[...], w1b_ref[...], preferred_element_type=jnp.float32,
                      precision=lax.Precision.DEFAULT)
            + b1_ref[...]
        )
        mu = jnp.mean(h, axis=-1, keepdims=True)
        var = jnp.mean((h - mu) ** 2, axis=-1, keepdims=True)
        h = (h - mu) * lax.rsqrt(var + 1e-5) * g1_ref[...] + be1_ref[...]
        h = jnp.tanh(h)
        o_ref[...] = (
            jnp.dot(h, w2_ref[...], preferred_element_type=jnp.float32,
                    precision=lax.Precision.DEFAULT)
            + b2_ref[...]
        )

    full = lambda shape: pl.BlockSpec(shape, lambda i: tuple(0 for _ in shape))
    return pl.pallas_call(
        body,
        grid=(N_PAD // BN,),
        in_specs=[
            pl.BlockSpec((NC, BN, D), lambda i: (0, i, 0)),
            pl.BlockSpec((BN, D), lambda i: (i, 0)),
            full((D, D)),
            full((D, D)),
            full((1, D)),
            full((1, D)),
            full((1, D)),
            full((D, D)),
            full((1, D)),
        ],
        out_specs=pl.BlockSpec((BN, D), lambda i: (i, 0)),
        out_shape=jax.ShapeDtypeStruct((N_NODES, D), jnp.float32),
    )(msgs, x_pad, W1a, W1b, b1, g1, be1, W2, b2)


def kernel(x, edge_index, W1, b1, g1, be1, W2, b2):
    x_pad = jnp.pad(x, ((0, N_PAD - N_NODES), (0, 0)))
    s = edge_index[0]
    e = edge_index[1]
    n_fill = E_PAD - 2 * N_EDGES
    fill = jnp.arange(n_fill, dtype=jnp.int32)
    src_fill = fill * 97 % N_NODES
    dst_fill = N_NODES + fill % (N_PAD - N_NODES)
    src = jnp.concatenate([s, e, src_fill]).reshape(NW, NR, RK, CHUNK)
    dst = jnp.concatenate([e, s, dst_fill]).reshape(NW, NR, RK, CHUNK)
    sdx = jnp.stack([src, dst], axis=2)
    zeros_tile = jnp.zeros((ROWS_PER_TILE, D), jnp.float32)

    msgs = _sc_messages(x_pad, sdx, zeros_tile)
    out = _tc_mlp(msgs, x_pad, W1[:D], W1[D:], b1.reshape(1, D),
                  g1.reshape(1, D), be1.reshape(1, D), W2, b2.reshape(1, D))
    return out
